# Initial kernel scaffold; baseline (speedup 1.0000x reference)
import jax
import jax.numpy as jnp
from jax.experimental import pallas as pl


def kernel(input_seq, weights):
    raise NotImplementedError("write your pallas kernel here")


# speedup vs baseline: 2.2904x; 2.2904x over previous
import jax
import jax.numpy as jnp
from jax.experimental import pallas as pl

_TILE = 512


def _bcast_body(w_ref, o_ref):
    o_ref[...] = jnp.broadcast_to(w_ref[...][None, :, :], o_ref.shape)


def kernel(input_seq, weights):
    batch, seq = input_seq.shape
    emb = weights.shape[1]
    tile = _TILE if seq % _TILE == 0 else seq
    return pl.pallas_call(
        _bcast_body,
        grid=(seq // tile,),
        in_specs=[pl.BlockSpec((tile, emb), lambda i: (i, 0))],
        out_specs=pl.BlockSpec((batch, tile, emb), lambda i: (0, i, 0)),
        out_shape=jax.ShapeDtypeStruct((batch, seq, emb), weights.dtype),
    )(weights)
